# baseline (speedup 1.0000x reference)
import functools

import jax
import jax.numpy as jnp
from jax.experimental import pallas as pl
from jax.experimental.pallas import tpu as pltpu

_TOP_K = 8


def _gate_body(x_ref, w_ref, sm_ref, idx_ref, acc_ref):
    k = pl.program_id(1)
    nk = pl.num_programs(1)
    part = jax.lax.dot_general(
        w_ref[...], x_ref[...],
        dimension_numbers=(((1,), (1,)), ((), ())),
        preferred_element_type=jnp.float32,
    )

    @pl.when(k == 0)
    def _():
        acc_ref[...] = part

    @pl.when(k > 0)
    def _():
        acc_ref[...] += part

    @pl.when(k == nk - 1)
    def _():
        logits = acc_ref[...]
        ne, bt = logits.shape
        row = jax.lax.broadcasted_iota(jnp.int32, (ne, bt), 0)
        bits = jax.lax.bitcast_convert_type(logits, jnp.int32)
        skey = bits ^ ((bits >> 31) & jnp.int32(0x7FFFFFFF))
        key = (skey & jnp.int32(~0x3F)) | (row ^ jnp.int32(0x3F))
        neg = jnp.int32(-(2**31))
        vals = key
        keys = []
        for _ in range(_TOP_K):
            m = jnp.max(vals, axis=0, keepdims=True)
            keys.append(m)
            vals = jnp.where(vals == m, neg, vals)
        k8 = jnp.concatenate(keys, axis=0).T
        idx = (k8 & jnp.int32(0x3F)) ^ jnp.int32(0x3F)
        st = k8 & jnp.int32(~0x3F)
        sbits = st ^ ((st >> 31) & jnp.int32(0x7FFFFFFF))
        s = jax.lax.bitcast_convert_type(sbits, jnp.float32)
        e = jnp.exp(s - s[:, 0:1])
        sm_ref[...] = e / jnp.sum(e, axis=1, keepdims=True)
        idx_ref[...] = idx


@functools.partial(jax.jit, static_argnames=("bt", "nk"))
def _gate(x, w, bt, nk):
    t, d = x.shape
    ne = w.shape[0]
    dk = d // nk
    return pl.pallas_call(
        _gate_body,
        grid=(t // bt, nk),
        in_specs=[
            pl.BlockSpec((bt, dk), lambda i, k: (i, k)),
            pl.BlockSpec((ne, dk), lambda i, k: (0, k)),
        ],
        out_specs=[
            pl.BlockSpec((bt, _TOP_K), lambda i, k: (i, 0)),
            pl.BlockSpec((bt, _TOP_K), lambda i, k: (i, 0)),
        ],
        out_shape=[
            jax.ShapeDtypeStruct((t, _TOP_K), jnp.float32),
            jax.ShapeDtypeStruct((t, _TOP_K), jnp.int32),
        ],
        scratch_shapes=[pltpu.VMEM((ne, bt), jnp.float32)],
    )(x, w)


def kernel(x, W):
    sm, idx = _gate(x, W, bt=2048, nk=4)
    return (sm, idx)


# speedup vs baseline: 1.0798x; 1.0798x over previous
import functools

import jax
import jax.numpy as jnp
from jax.experimental import pallas as pl
from jax.experimental.pallas import tpu as pltpu

_TOP_K = 8


def _gate_body(x_ref, w_ref, sm_ref, idx_ref, acc_ref):
    k = pl.program_id(1)
    nk = pl.num_programs(1)
    part = jax.lax.dot_general(
        w_ref[...], x_ref[...],
        dimension_numbers=(((1,), (1,)), ((), ())),
        preferred_element_type=jnp.float32,
    )

    @pl.when(k == 0)
    def _():
        acc_ref[...] = part

    @pl.when(k > 0)
    def _():
        acc_ref[...] += part

    @pl.when(k == nk - 1)
    def _():
        logits = acc_ref[...]
        ne, bt = logits.shape
        row = jax.lax.broadcasted_iota(jnp.int32, (ne, bt), 0)
        bits = jax.lax.bitcast_convert_type(logits, jnp.int32)
        skey = bits ^ ((bits >> 31) & jnp.int32(0x7FFFFFFF))
        key = (skey & jnp.int32(~0x3F)) | (row ^ jnp.int32(0x3F))
        neg = jnp.int32(-(2**31))
        vals = key
        keys = []
        for _ in range(_TOP_K):
            m = jnp.max(vals, axis=0, keepdims=True)
            keys.append(m)
            vals = jnp.where(vals == m, neg, vals)
        k8 = jnp.concatenate(keys, axis=0).T
        idx = (k8 & jnp.int32(0x3F)) ^ jnp.int32(0x3F)
        st = k8 & jnp.int32(~0x3F)
        sbits = st ^ ((st >> 31) & jnp.int32(0x7FFFFFFF))
        s = jax.lax.bitcast_convert_type(sbits, jnp.float32)
        e = jnp.exp(s - s[:, 0:1])
        sm_ref[...] = e / jnp.sum(e, axis=1, keepdims=True)
        idx_ref[...] = idx


@functools.partial(jax.jit, static_argnames=("bt", "nk"))
def _gate(x, w, bt, nk):
    t, d = x.shape
    ne = w.shape[0]
    dk = d // nk
    return pl.pallas_call(
        _gate_body,
        grid=(t // bt, nk),
        in_specs=[
            pl.BlockSpec((bt, dk), lambda i, k: (i, k)),
            pl.BlockSpec((ne, dk), lambda i, k: (0, k)),
        ],
        out_specs=[
            pl.BlockSpec((bt, _TOP_K), lambda i, k: (i, 0)),
            pl.BlockSpec((bt, _TOP_K), lambda i, k: (i, 0)),
        ],
        out_shape=[
            jax.ShapeDtypeStruct((t, _TOP_K), jnp.float32),
            jax.ShapeDtypeStruct((t, _TOP_K), jnp.int32),
        ],
        scratch_shapes=[pltpu.VMEM((ne, bt), jnp.float32)],
    )(x, w)


def kernel(x, W):
    sm, idx = _gate(x, W, bt=2048, nk=2)
    return (sm, idx)


# speedup vs baseline: 1.1493x; 1.0643x over previous
import functools

import jax
import jax.numpy as jnp
from jax.experimental import pallas as pl
from jax.experimental.pallas import tpu as pltpu

_TOP_K = 8


def _gate_body(x_ref, w_ref, sm_ref, idx_ref, acc_ref):
    k = pl.program_id(1)
    nk = pl.num_programs(1)
    part = jax.lax.dot_general(
        w_ref[...], x_ref[...],
        dimension_numbers=(((1,), (1,)), ((), ())),
        preferred_element_type=jnp.float32,
    )

    @pl.when(k == 0)
    def _():
        acc_ref[...] = part

    @pl.when(k > 0)
    def _():
        acc_ref[...] += part

    @pl.when(k == nk - 1)
    def _():
        logits = acc_ref[...]
        ne, bt = logits.shape
        row = jax.lax.broadcasted_iota(jnp.int32, (ne, bt), 0)
        bits = jax.lax.bitcast_convert_type(logits, jnp.int32)
        skey = bits ^ ((bits >> 31) & jnp.int32(0x7FFFFFFF))
        key = (skey & jnp.int32(~0x3F)) | (row ^ jnp.int32(0x3F))
        neg = jnp.int32(-(2**31))
        vals = key
        keys = []
        for _ in range(_TOP_K):
            m = jnp.max(vals, axis=0, keepdims=True)
            keys.append(m)
            vals = jnp.where(vals == m, neg, vals)
        k8 = jnp.concatenate(keys, axis=0).T
        idx = (k8 & jnp.int32(0x3F)) ^ jnp.int32(0x3F)
        st = k8 & jnp.int32(~0x3F)
        sbits = st ^ ((st >> 31) & jnp.int32(0x7FFFFFFF))
        s = jax.lax.bitcast_convert_type(sbits, jnp.float32)
        e = jnp.exp(s - s[:, 0:1])
        sm_ref[...] = e / jnp.sum(e, axis=1, keepdims=True)
        idx_ref[...] = idx


@functools.partial(jax.jit, static_argnames=("bt", "nk"))
def _gate(x, w, bt, nk):
    t, d = x.shape
    ne = w.shape[0]
    dk = d // nk
    return pl.pallas_call(
        _gate_body,
        grid=(t // bt, nk),
        in_specs=[
            pl.BlockSpec((bt, dk), lambda i, k: (i, k)),
            pl.BlockSpec((ne, dk), lambda i, k: (0, k)),
        ],
        out_specs=[
            pl.BlockSpec((bt, _TOP_K), lambda i, k: (i, 0)),
            pl.BlockSpec((bt, _TOP_K), lambda i, k: (i, 0)),
        ],
        out_shape=[
            jax.ShapeDtypeStruct((t, _TOP_K), jnp.float32),
            jax.ShapeDtypeStruct((t, _TOP_K), jnp.int32),
        ],
        scratch_shapes=[pltpu.VMEM((ne, bt), jnp.float32)],
    )(x, w)


def kernel(x, W):
    sm, idx = _gate(x, W, bt=1024, nk=1)
    return (sm, idx)
